# Initial kernel scaffold; baseline (speedup 1.0000x reference)
import jax
import jax.numpy as jnp
from jax.experimental import pallas as pl


def kernel(inputs, emb_weight):
    raise NotImplementedError("write your pallas kernel here")


# speedup vs baseline: 1.0947x; 1.0947x over previous
import jax
import jax.numpy as jnp
from jax import lax
from jax.experimental import pallas as pl
from jax.experimental.pallas import tpu as pltpu, tpu_sc as plsc

NUM_CORES = 2
NUM_SUBCORES = 16
NW = NUM_CORES * NUM_SUBCORES
BATCH = 16384
HIST = 50
EMB = 32
B_TOTAL = BATCH * HIST
B_PER_W = B_TOTAL // NW
CHUNK = 1024
N_CHUNKS = B_PER_W // CHUNK


def _gather_body(idx_hbm, table_hbm, out_hbm, idx_v, rows_v, sem):
    wid = lax.axis_index("s") * NUM_CORES + lax.axis_index("c")
    base = wid * B_PER_W

    def body(i, carry):
        start = base + i * CHUNK
        pltpu.sync_copy(idx_hbm.at[pl.ds(start, CHUNK)], idx_v)
        pltpu.async_copy(table_hbm.at[idx_v], rows_v, sem).wait()
        pltpu.sync_copy(rows_v, out_hbm.at[pl.ds(start, CHUNK)])
        return carry

    lax.fori_loop(0, N_CHUNKS, body, 0)


@jax.jit
def kernel(inputs, emb_weight):
    idx = inputs.reshape(-1).astype(jnp.int32)
    mesh = plsc.VectorSubcoreMesh(
        core_axis_name="c", subcore_axis_name="s",
        num_cores=NUM_CORES, num_subcores=NUM_SUBCORES)
    out = pl.kernel(
        _gather_body,
        out_type=jax.ShapeDtypeStruct((B_TOTAL, EMB), jnp.float32),
        mesh=mesh,
        compiler_params=pltpu.CompilerParams(use_tc_tiling_on_sc=False),
        scratch_types=[
            pltpu.VMEM((CHUNK,), jnp.int32),
            pltpu.VMEM((CHUNK, EMB), jnp.float32),
            pltpu.SemaphoreType.DMA,
        ],
    )(idx, emb_weight)
    return out.reshape(BATCH, HIST, EMB)


# speedup vs baseline: 1.1135x; 1.0172x over previous
import jax
import jax.numpy as jnp
from jax import lax
from jax.experimental import pallas as pl
from jax.experimental.pallas import tpu as pltpu, tpu_sc as plsc

NUM_CORES = 2
NUM_SUBCORES = 16
NW = NUM_CORES * NUM_SUBCORES
BATCH = 16384
HIST = 50
EMB = 32
B_TOTAL = BATCH * HIST
B_PER_W = B_TOTAL // NW
CHUNK = 1280
N_CHUNKS = B_PER_W // CHUNK


def _gather_body(idx_hbm, table_hbm, out_hbm,
                 idx_all, rows0, rows1, sg0, sg1, so0, so1):
    wid = lax.axis_index("s") * NUM_CORES + lax.axis_index("c")
    base = wid * B_PER_W

    rows = (rows0, rows1)
    sg = (sg0, sg1)
    so = (so0, so1)

    pltpu.sync_copy(idx_hbm.at[pl.ds(base, B_PER_W)], idx_all)

    def gat(k, b):
        pltpu.async_copy(table_hbm.at[idx_all.at[pl.ds(k * CHUNK, CHUNK)]],
                         rows[b], sg[b])

    def wait_g(b):
        pltpu.make_async_copy(table_hbm.at[idx_all.at[pl.ds(0, CHUNK)]],
                              rows[b], sg[b]).wait()

    def sto(k, b):
        pltpu.async_copy(rows[b], out_hbm.at[pl.ds(base + k * CHUNK, CHUNK)],
                         so[b])

    def wait_s(b):
        pltpu.make_async_copy(rows[b], out_hbm.at[pl.ds(base, CHUNK)],
                              so[b]).wait()

    gat(0, 0)
    gat(1, 1)
    wait_g(0)
    sto(0, 0)

    def pair(i, carry):
        kk = 1 + 2 * i
        for off in (0, 1):
            k = kk + off
            b = (1 + off) & 1
            o = 1 - b
            wait_s(o)
            gat(k + 1, o)
            wait_g(b)
            sto(k, b)
        return carry

    lax.fori_loop(0, (N_CHUNKS - 2) // 2, pair, 0)

    wait_g(1)
    sto(N_CHUNKS - 1, 1)
    wait_s(0)
    wait_s(1)


@jax.jit
def kernel(inputs, emb_weight):
    idx = inputs.reshape(-1).astype(jnp.int32)
    mesh = plsc.VectorSubcoreMesh(
        core_axis_name="c", subcore_axis_name="s",
        num_cores=NUM_CORES, num_subcores=NUM_SUBCORES)
    out = pl.kernel(
        _gather_body,
        out_type=jax.ShapeDtypeStruct((B_TOTAL, EMB), jnp.float32),
        mesh=mesh,
        compiler_params=pltpu.CompilerParams(use_tc_tiling_on_sc=False),
        scratch_types=[
            pltpu.VMEM((B_PER_W,), jnp.int32),
            pltpu.VMEM((CHUNK, EMB), jnp.float32),
            pltpu.VMEM((CHUNK, EMB), jnp.float32),
            pltpu.SemaphoreType.DMA,
            pltpu.SemaphoreType.DMA,
            pltpu.SemaphoreType.DMA,
            pltpu.SemaphoreType.DMA,
        ],
    )(idx, emb_weight)
    return out.reshape(BATCH, HIST, EMB)


# speedup vs baseline: 1.8089x; 1.6245x over previous
import jax
import jax.numpy as jnp
from jax import lax
from jax.experimental import pallas as pl
from jax.experimental.pallas import tpu as pltpu, tpu_sc as plsc

NUM_CORES = 2
NUM_SUBCORES = 16
NW = NUM_CORES * NUM_SUBCORES
BATCH = 16384
HIST = 50
EMB = 32
B_TOTAL = BATCH * HIST
B_PER_W = B_TOTAL // NW
CHUNK = 1600
CB = CHUNK // HIST
N_CHUNKS = B_PER_W // CHUNK


def _gather_body(idx_hbm, table_hbm, out_hbm,
                 idx_all, rows0, rows1, sg0, sg1, so0, so1):
    wid = lax.axis_index("s") * NUM_CORES + lax.axis_index("c")
    base = wid * B_PER_W
    base_b = wid * (B_PER_W // HIST)

    rows = (rows0, rows1)
    sg = (sg0, sg1)
    so = (so0, so1)

    pltpu.sync_copy(idx_hbm.at[pl.ds(base, B_PER_W)], idx_all)

    def gat(k, b):
        pltpu.async_copy(table_hbm.at[idx_all.at[pl.ds(k * CHUNK, CHUNK)]],
                         rows[b], sg[b])

    def wait_g(b):
        pltpu.make_async_copy(table_hbm.at[idx_all.at[pl.ds(0, CHUNK)]],
                              rows[b], sg[b]).wait()

    def sto(k, b):
        for j in range(CB):
            pltpu.async_copy(rows[b].at[pl.ds(j * HIST, HIST)],
                             out_hbm.at[base_b + k * CB + j],
                             so[b])

    def wait_s(b):
        for j in range(CB):
            pltpu.make_async_copy(rows[b].at[pl.ds(0, HIST)],
                                  out_hbm.at[base_b],
                                  so[b]).wait()

    gat(0, 0)
    gat(1, 1)
    wait_g(0)
    sto(0, 0)

    def pair(i, carry):
        kk = 1 + 2 * i
        for off in (0, 1):
            k = kk + off
            b = (1 + off) & 1
            o = 1 - b
            wait_s(o)
            gat(k + 1, o)
            wait_g(b)
            sto(k, b)
        return carry

    lax.fori_loop(0, (N_CHUNKS - 2) // 2, pair, 0)

    wait_g(1)
    sto(N_CHUNKS - 1, 1)
    wait_s(0)
    wait_s(1)


@jax.jit
def kernel(inputs, emb_weight):
    idx = inputs.reshape(-1).astype(jnp.int32)
    mesh = plsc.VectorSubcoreMesh(
        core_axis_name="c", subcore_axis_name="s",
        num_cores=NUM_CORES, num_subcores=NUM_SUBCORES)
    out = pl.kernel(
        _gather_body,
        out_type=jax.ShapeDtypeStruct((BATCH, HIST, EMB), jnp.float32),
        mesh=mesh,
        compiler_params=pltpu.CompilerParams(use_tc_tiling_on_sc=False),
        scratch_types=[
            pltpu.VMEM((B_PER_W,), jnp.int32),
            pltpu.VMEM((CHUNK, EMB), jnp.float32),
            pltpu.VMEM((CHUNK, EMB), jnp.float32),
            pltpu.SemaphoreType.DMA,
            pltpu.SemaphoreType.DMA,
            pltpu.SemaphoreType.DMA,
            pltpu.SemaphoreType.DMA,
        ],
    )(idx, emb_weight)
    return out
